# baseline (speedup 1.0000x reference)
import functools

import jax
import jax.numpy as jnp
from jax import lax
from jax.experimental import pallas as pl
from jax.experimental.pallas import tpu as pltpu
from jax.experimental.pallas import tpu_sc as plsc

B, N, H, W, D = 128, 10, 30, 30, 64
NUM_COLORS = 11
HW = H * W
P = N * NUM_COLORS
CELLS = B * N * HW
NW = 32
CPT = CELLS // NW
CHUNK = 120
KCH = 4
SUPER = CHUNK * KCH
ITERS = CPT // SUPER
NBUF = 3
PPT = 7
HALF = HW // 2

_MESH = plsc.VectorSubcoreMesh(core_axis_name="c", subcore_axis_name="s")


@functools.partial(
    pl.kernel,
    mesh=_MESH,
    compiler_params=pltpu.CompilerParams(use_tc_tiling_on_sc=False),
    out_type=(
        jax.ShapeDtypeStruct((CELLS, D), jnp.float32),
        jax.ShapeDtypeStruct((2 * P * HW, D), jnp.float32),
    ),
    scratch_types=[
        pltpu.VMEM((P, D), jnp.float32),
        pltpu.VMEM((NBUF, SUPER), jnp.int32),
        pltpu.VMEM((NBUF, SUPER, D), jnp.float32),
        pltpu.VMEM((SUPER,), jnp.int32),
        pltpu.VMEM((SUPER,), jnp.int32),
        pltpu.SemaphoreType.DMA,
        pltpu.SemaphoreType.DMA,
        pltpu.SemaphoreType.DMA,
        pltpu.SemaphoreType.DMA,
        pltpu.SemaphoreType.DMA,
        pltpu.SemaphoreType.DMA,
    ],
)
def _sc_embed(g_hbm, a_hbm, fused_hbm, rowcol_hbm, out_hbm, mega_hbm,
              fusedv, idx_v, rows_v, gtmp, atmp,
              gs0, gs1, gs2, ss0, ss1, ss2):
    gsem = (gs0, gs1, gs2)
    ssem = (ss0, ss1, ss2)
    cid = lax.axis_index("c")
    sid = lax.axis_index("s")
    wid = sid * 2 + cid
    mrow0 = cid * (P * HW)

    pltpu.sync_copy(fused_hbm, fusedv)
    base_p = sid * PPT
    cnt = jnp.minimum(PPT, P - base_p)

    def build_p(k, carry):
        p = base_p + k

        def half_body(half, c2):
            pltpu.sync_copy(rowcol_hbm.at[pl.ds(half * HALF, HALF)],
                            rows_v.at[1, pl.ds(0, HALF)])

            def row_body(r, c3):
                for q in range(D // 16):
                    sl = pl.ds(q * 16, 16)
                    rows_v[0, r, sl] = rows_v[1, r, sl] + fusedv[p, sl]
                return c3

            lax.fori_loop(0, HALF, row_body, 0)
            pltpu.sync_copy(
                rows_v.at[0, pl.ds(0, HALF)],
                mega_hbm.at[pl.ds(mrow0 + p * HW + half * HALF, HALF)])
            return c2

        lax.fori_loop(0, 2, half_body, 0)
        return carry

    lax.fori_loop(0, cnt, build_p, 0)
    plsc.subcore_barrier()

    cell0 = wid * CPT

    def load_and_fire(i, b):
        base = cell0 + i * SUPER
        pltpu.sync_copy(g_hbm.at[pl.ds(base, SUPER)], gtmp)
        pltpu.sync_copy(a_hbm.at[pl.ds(base, SUPER)], atmp)
        for q in range(SUPER // 16):
            sl = pl.ds(q * 16, 16)
            idx_v[b, sl] = gtmp[sl] * HW + atmp[sl] + mrow0
        for j in range(KCH):
            pltpu.async_copy(
                mega_hbm.at[idx_v.at[b, pl.ds(j * CHUNK, CHUNK)]],
                rows_v.at[b, pl.ds(j * CHUNK, CHUNK)],
                gsem[b],
            )

    def drain_gathers(b):
        pltpu.make_async_copy(out_hbm.at[pl.ds(0, SUPER)], rows_v.at[b],
                              gsem[b]).wait()

    def fire_scatter(i, b):
        pltpu.async_copy(rows_v.at[b],
                         out_hbm.at[pl.ds(cell0 + i * SUPER, SUPER)], ssem[b])

    def wait_scatter(b):
        pltpu.make_async_copy(out_hbm.at[pl.ds(0, SUPER)], rows_v.at[b],
                              ssem[b]).wait()

    load_and_fire(0, 0)
    load_and_fire(1, 1)
    drain_gathers(0)
    fire_scatter(0, 0)
    load_and_fire(2, 2)

    def body(k, carry):
        for j in range(3):
            i = 1 + 3 * k + j
            b = (1 + j) % 3
            drain_gathers(b)
            fire_scatter(i, b)
            wait_scatter(j)
            load_and_fire(i + 2, j)
        return carry

    lax.fori_loop(0, (ITERS - 3) // 3, body, 0)

    for i in (ITERS - 2, ITERS - 1):
        b = i % 3
        drain_gathers(b)
        fire_scatter(i, b)
    for b in range(3):
        wait_scatter(b)


def kernel(grids, color_table, row_table, col_table, example_table, role_table):
    grids_flat = grids.astype(jnp.int32).reshape(CELLS)
    cell = jnp.arange(CELLS, dtype=jnp.int32)
    aconst = (cell // HW % N) * (NUM_COLORS * HW) + cell % HW
    eids = jnp.arange(N, dtype=jnp.int32) // 2 + 1
    rids = jnp.arange(N, dtype=jnp.int32) % 2
    exrole = example_table[eids] + role_table[rids]
    fused = (exrole[:, None, :] + color_table[None, :, :]).reshape(P, D)
    rowcol = (row_table[:, None, :] + col_table[None, :, :]).reshape(HW, D)
    out, _ = _sc_embed(grids_flat, aconst, fused, rowcol)
    return out.reshape(B, N, H, W, D)
